# baseline (speedup 1.0000x reference)
import functools

import numpy as np

import jax
import jax.numpy as jnp
from jax import lax
from jax.experimental import pallas as pl
from jax.experimental.pallas import tpu as pltpu
from jax.experimental.pallas import tpu_sc as plsc

N = 10000
E = 320000
NC = 2
NS = 16
NW = NC * NS
CHUNK = 128
C = 80
PH = 40
TOTC = NW * C
EP = TOTC * CHUNK
NP = 10240
RPT = NP // NS
DUMMY = 10016
CNTW = 16
ROWT = 400
GRID = N // ROWT


def _make_agg_bf(D):
  mesh = plsc.VectorSubcoreMesh(core_axis_name="c", subcore_axis_name="s")
  W = D // 2
  G = D // 32

  def convert(src_i, dst_f):
    def rows(i, carry):
      for u in range(4):
        r = 4 * i + u
        for g in range(G):
          w = src_i[r, pl.ds(16 * g, 16)]
          dst_f[r, pl.ds(32 * g, 16)] = plsc.bitcast(w << 16, jnp.float32)
          dst_f[r, pl.ds(32 * g + 16, 16)] = plsc.bitcast(
              w & jnp.int32(-65536), jnp.float32)
      return carry

    lax.fori_loop(0, CHUNK // 4, rows, 0)

  def body(table, srcp, dstp, zrows, out, src_v, dst_v, rows_i0, rows_i1,
           rows_f, acc_sh, sem0, sem1):
    c = lax.axis_index("c")
    s = lax.axis_index("s")
    r0 = s * RPT
    pltpu.sync_copy(zrows.at[pl.ds(r0, RPT)], acc_sh.at[pl.ds(r0, RPT)])
    plsc.subcore_barrier()

    base = (c * NS + s) * C
    for phase in range(C // PH):
      ph0 = base + phase * PH
      pltpu.sync_copy(srcp.at[pl.ds(ph0, PH)], src_v)
      pltpu.sync_copy(dstp.at[pl.ds(ph0, PH)], dst_v)
      pltpu.async_copy(table.at[src_v.at[0]], rows_i0, sem0)

      def step(i, carry):
        j0 = 2 * i
        j1 = j0 + 1
        j2 = lax.rem(j0 + 2, PH)
        pltpu.make_async_copy(table.at[src_v.at[j0]], rows_i0, sem0).wait()
        pltpu.async_copy(table.at[src_v.at[j1]], rows_i1, sem1)
        convert(rows_i0, rows_f)
        pltpu.sync_copy(rows_f, acc_sh.at[dst_v.at[j0]], add=True)
        pltpu.make_async_copy(table.at[src_v.at[j1]], rows_i1, sem1).wait()
        pltpu.async_copy(table.at[src_v.at[j2]], rows_i0, sem0)
        convert(rows_i1, rows_f)
        pltpu.sync_copy(rows_f, acc_sh.at[dst_v.at[j1]], add=True)
        return carry

      lax.fori_loop(0, PH // 2, step, 0)
      pltpu.make_async_copy(table.at[src_v.at[0]], rows_i0, sem0).wait()
    plsc.subcore_barrier()
    pltpu.sync_copy(acc_sh.at[pl.ds(r0, RPT)], out.at[c, pl.ds(r0, RPT)])

  return pl.kernel(
      body,
      out_type=jax.ShapeDtypeStruct((NC, NP, D), jnp.float32),
      mesh=mesh,
      compiler_params=pltpu.CompilerParams(use_tc_tiling_on_sc=False,
                                           needs_layout_passes=False),
      scratch_types=(
          pltpu.VMEM((PH, CHUNK), jnp.int32),
          pltpu.VMEM((PH, CHUNK), jnp.int32),
          pltpu.VMEM((CHUNK, W), jnp.int32),
          pltpu.VMEM((CHUNK, W), jnp.int32),
          pltpu.VMEM((CHUNK, D), jnp.float32),
          pltpu.VMEM_SHARED((NP, D), jnp.float32),
          pltpu.SemaphoreType.DMA,
          pltpu.SemaphoreType.DMA,
      ))


def _sigma(D):
  o = np.concatenate([
      np.concatenate([32 * g + np.arange(0, 32, 2),
                      32 * g + np.arange(1, 32, 2)])
      for g in range(D // 32)
  ])
  s = np.empty(D, np.int64)
  s[o] = np.arange(D)
  return s


_SIG128 = _sigma(128)
_SIG64 = _sigma(64)


def _pack_bf16(t):
  sig = _SIG128 if t.shape[1] == 128 else _SIG64
  tb = t[:, sig].astype(jnp.bfloat16)
  return jax.lax.bitcast_convert_type(
      tb.reshape(t.shape[0], t.shape[1] // 2, 2), jnp.int32)


def _make_cnt():
  mesh = plsc.VectorSubcoreMesh(core_axis_name="c", subcore_axis_name="s")

  def body(dstp, zcnt, ones_h, cnt_out, dst_v, ones_v, cnt_sh):
    c = lax.axis_index("c")
    s = lax.axis_index("s")
    r0 = s * RPT
    pltpu.sync_copy(zcnt.at[pl.ds(r0, RPT)], cnt_sh.at[pl.ds(r0, RPT)])
    pltpu.sync_copy(ones_h, ones_v)
    pltpu.sync_copy(dstp.at[pl.ds((c * NS + s) * C, C)], dst_v)
    plsc.subcore_barrier()

    def step(j, carry):
      pltpu.sync_copy(ones_v, cnt_sh.at[dst_v.at[j]], add=True)
      return carry

    lax.fori_loop(0, C, step, 0)
    plsc.subcore_barrier()
    pltpu.sync_copy(cnt_sh.at[pl.ds(r0, RPT)], cnt_out.at[c, pl.ds(r0, RPT)])

  return pl.kernel(
      body,
      out_type=jax.ShapeDtypeStruct((NC, NP, CNTW), jnp.float32),
      mesh=mesh,
      compiler_params=pltpu.CompilerParams(use_tc_tiling_on_sc=False),
      scratch_types=(
          pltpu.VMEM((C, CHUNK), jnp.int32),
          pltpu.VMEM((CHUNK, CNTW), jnp.float32),
          pltpu.VMEM_SHARED((NP, CNTW), jnp.float32),
      ))


_agg_bf_128 = _make_agg_bf(128)
_agg_bf_64 = _make_agg_bf(64)
_cnt_call = _make_cnt()


def _inv_deg(cnt_ref):
  cv = cnt_ref[0] + cnt_ref[1]
  return 1.0 / jnp.maximum(jnp.max(cv, axis=1, keepdims=True), 1.0)


def _dot(a, b):
  return jnp.dot(a, b, preferred_element_type=jnp.float32)


def _tc1_body(agg, cnt, xb, wl1, wr1, b1, wl2, wr2, b2, p2, r2):
  mean = (agg[0] + agg[1]) * _inv_deg(cnt)
  h = _dot(mean, wl1[...]) + _dot(xb[...], wr1[...]) + b1[...]
  h = jnp.maximum(h, 0.0)
  p2[...] = _dot(h, wl2[...])
  r2[...] = _dot(h, wr2[...]) + b2[...]


def _tc2_body(agg, cnt, r2, wl3, wr3, b3, p3, r3):
  h = jnp.maximum((agg[0] + agg[1]) * _inv_deg(cnt) + r2[...], 0.0)
  p3[...] = _dot(h, wl3[...])
  r3[...] = _dot(h, wr3[...]) + b3[...]


def _tc3_body(agg, cnt, r3, out, ls):
  o = (agg[0] + agg[1]) * _inv_deg(cnt) + r3[...]
  out[...] = o
  m = jnp.max(o, axis=1, keepdims=True)
  e = jnp.exp(o - m)
  ls[...] = o - m - jnp.log(jnp.sum(e, axis=1, keepdims=True))


def _row_spec(d):
  return pl.BlockSpec((ROWT, d), lambda i: (i, 0))


def _agg_spec(d):
  return pl.BlockSpec((NC, ROWT, d), lambda i: (0, i, 0))


def _full_spec(shape):
  return pl.BlockSpec(shape, lambda i: tuple(0 for _ in shape))


def _tc_call(body, in_specs, out_specs, out_shape):
  return pl.pallas_call(
      body, grid=(GRID,), in_specs=in_specs, out_specs=out_specs,
      out_shape=out_shape)


def kernel(x, edge_index, Wl1, Wr1, b1, Wl2, Wr2, b2, Wl3, Wr3, b3):
  src = edge_index[0].astype(jnp.int32)
  dst = edge_index[1].astype(jnp.int32)
  pad = EP - E
  src_p = jnp.concatenate([src, jnp.zeros((pad,), jnp.int32)])
  dst_p = jnp.concatenate([dst, jnp.full((pad,), DUMMY, jnp.int32)])
  src_p = src_p.reshape(TOTC, CHUNK)
  dst_p = dst_p.reshape(TOTC, CHUNK)

  z128 = jnp.zeros((NP, 128), jnp.float32)
  z64 = jnp.zeros((NP, 64), jnp.float32)
  zcnt = jnp.zeros((NP, CNTW), jnp.float32)
  ones_h = jnp.ones((CHUNK, CNTW), jnp.float32)
  b1r = b1.reshape(1, -1)
  b2r = b2.reshape(1, -1)
  b3r = b3.reshape(1, -1)

  cnt = _cnt_call(dst_p, zcnt, ones_h)
  agg1 = _agg_bf_128(_pack_bf16(x), src_p, dst_p, z128)

  p2, r2 = _tc_call(
      _tc1_body,
      [_agg_spec(128), _agg_spec(CNTW), _row_spec(128),
       _full_spec((128, 256)), _full_spec((128, 256)), _full_spec((1, 256)),
       _full_spec((256, 128)), _full_spec((256, 128)), _full_spec((1, 128))],
      [_row_spec(128), _row_spec(128)],
      [jax.ShapeDtypeStruct((N, 128), jnp.float32),
       jax.ShapeDtypeStruct((N, 128), jnp.float32)],
  )(agg1, cnt, x, Wl1, Wr1, b1r, Wl2, Wr2, b2r)

  agg2 = _agg_bf_128(_pack_bf16(p2), src_p, dst_p, z128)

  p3, r3 = _tc_call(
      _tc2_body,
      [_agg_spec(128), _agg_spec(CNTW), _row_spec(128),
       _full_spec((128, 64)), _full_spec((128, 64)), _full_spec((1, 64))],
      [_row_spec(64), _row_spec(64)],
      [jax.ShapeDtypeStruct((N, 64), jnp.float32),
       jax.ShapeDtypeStruct((N, 64), jnp.float32)],
  )(agg2, cnt, r2, Wl3, Wr3, b3r)

  agg3 = _agg_bf_64(_pack_bf16(p3), src_p, dst_p, z64)

  out, ls = _tc_call(
      _tc3_body,
      [_agg_spec(64), _agg_spec(CNTW), _row_spec(64)],
      [_row_spec(64), _row_spec(64)],
      [jax.ShapeDtypeStruct((N, 64), jnp.float32),
       jax.ShapeDtypeStruct((N, 64), jnp.float32)],
  )(agg3, cnt, r3)

  return (out, ls)


# speedup vs baseline: 1.0050x; 1.0050x over previous
import numpy as np

import jax
import jax.numpy as jnp
from jax import lax
from jax.experimental import pallas as pl
from jax.experimental.pallas import tpu as pltpu
from jax.experimental.pallas import tpu_sc as plsc

N = 10000
E = 320000
NC = 2
NS = 16
NW = NC * NS
CHUNK = 128
C = 80
PH = 40
TOTC = NW * C
EP = TOTC * CHUNK
NP = 10240
RPT = NP // NS
DUMMY = 10016
CNTW = 16
ROWT = 400
GRID = N // ROWT


def _make_agg_bf(D):
  mesh = plsc.VectorSubcoreMesh(core_axis_name="c", subcore_axis_name="s")
  W = D // 2
  G = D // 32

  def convert(src_i, dst_f):
    def rows(i, carry):
      for u in range(4):
        r = 4 * i + u
        for g in range(G):
          w = src_i[r, pl.ds(16 * g, 16)]
          dst_f[r, pl.ds(32 * g, 16)] = plsc.bitcast(w << 16, jnp.float32)
          dst_f[r, pl.ds(32 * g + 16, 16)] = plsc.bitcast(
              w & jnp.int32(-65536), jnp.float32)
      return carry

    lax.fori_loop(0, CHUNK // 4, rows, 0)

  def body(table, srcp, dstp, zrows, out, src_v, dst_v, rows_i0, rows_i1,
           rows_f, acc_sh, sem0, sem1):
    c = lax.axis_index("c")
    s = lax.axis_index("s")
    r0 = s * RPT
    pltpu.sync_copy(zrows.at[pl.ds(r0, RPT)], acc_sh.at[pl.ds(r0, RPT)])
    plsc.subcore_barrier()

    base = (c * NS + s) * C
    for phase in range(C // PH):
      ph0 = base + phase * PH
      pltpu.sync_copy(srcp.at[pl.ds(ph0, PH)], src_v)
      pltpu.sync_copy(dstp.at[pl.ds(ph0, PH)], dst_v)
      pltpu.async_copy(table.at[src_v.at[0]], rows_i0, sem0)

      def step(i, carry):
        j0 = 2 * i
        j1 = j0 + 1
        j2 = lax.rem(j0 + 2, PH)
        pltpu.make_async_copy(table.at[src_v.at[j0]], rows_i0, sem0).wait()
        pltpu.async_copy(table.at[src_v.at[j1]], rows_i1, sem1)
        convert(rows_i0, rows_f)
        pltpu.sync_copy(rows_f, acc_sh.at[dst_v.at[j0]], add=True)
        pltpu.make_async_copy(table.at[src_v.at[j1]], rows_i1, sem1).wait()
        pltpu.async_copy(table.at[src_v.at[j2]], rows_i0, sem0)
        convert(rows_i1, rows_f)
        pltpu.sync_copy(rows_f, acc_sh.at[dst_v.at[j1]], add=True)
        return carry

      lax.fori_loop(0, PH // 2, step, 0)
      pltpu.make_async_copy(table.at[src_v.at[0]], rows_i0, sem0).wait()
    plsc.subcore_barrier()
    pltpu.sync_copy(acc_sh.at[pl.ds(r0, RPT)], out.at[c, pl.ds(r0, RPT)])

  return pl.kernel(
      body,
      out_type=jax.ShapeDtypeStruct((NC, NP, D), jnp.float32),
      mesh=mesh,
      compiler_params=pltpu.CompilerParams(use_tc_tiling_on_sc=False,
                                           needs_layout_passes=False),
      scratch_types=(
          pltpu.VMEM((PH, CHUNK), jnp.int32),
          pltpu.VMEM((PH, CHUNK), jnp.int32),
          pltpu.VMEM((CHUNK, W), jnp.int32),
          pltpu.VMEM((CHUNK, W), jnp.int32),
          pltpu.VMEM((CHUNK, D), jnp.float32),
          pltpu.VMEM_SHARED((NP, D), jnp.float32),
          pltpu.SemaphoreType.DMA,
          pltpu.SemaphoreType.DMA,
      ))


def _sigma(D):
  o = np.concatenate([
      np.concatenate([32 * g + np.arange(0, 32, 2),
                      32 * g + np.arange(1, 32, 2)])
      for g in range(D // 32)
  ])
  s = np.empty(D, np.int64)
  s[o] = np.arange(D)
  return s


_SIG128 = _sigma(128)
_SIG64 = _sigma(64)


def _pack_bf16(t):
  sig = _SIG128 if t.shape[1] == 128 else _SIG64
  tb = t[:, sig].astype(jnp.bfloat16)
  return jax.lax.bitcast_convert_type(
      tb.reshape(t.shape[0], t.shape[1] // 2, 2), jnp.int32)


def _make_cnt():
  mesh = plsc.VectorSubcoreMesh(core_axis_name="c", subcore_axis_name="s")

  def body(dstp, zcnt, ones_h, cnt_out, dst_v, ones_v, cnt_sh):
    c = lax.axis_index("c")
    s = lax.axis_index("s")
    r0 = s * RPT
    pltpu.sync_copy(zcnt.at[pl.ds(r0, RPT)], cnt_sh.at[pl.ds(r0, RPT)])
    pltpu.sync_copy(ones_h, ones_v)
    pltpu.sync_copy(dstp.at[pl.ds((c * NS + s) * C, C)], dst_v)
    plsc.subcore_barrier()

    def step(j, carry):
      pltpu.sync_copy(ones_v, cnt_sh.at[dst_v.at[j]], add=True)
      return carry

    lax.fori_loop(0, C, step, 0)
    plsc.subcore_barrier()
    pltpu.sync_copy(cnt_sh.at[pl.ds(r0, RPT)], cnt_out.at[c, pl.ds(r0, RPT)])

  return pl.kernel(
      body,
      out_type=jax.ShapeDtypeStruct((NC, NP, CNTW), jnp.float32),
      mesh=mesh,
      compiler_params=pltpu.CompilerParams(use_tc_tiling_on_sc=False),
      scratch_types=(
          pltpu.VMEM((C, CHUNK), jnp.int32),
          pltpu.VMEM((CHUNK, CNTW), jnp.float32),
          pltpu.VMEM_SHARED((NP, CNTW), jnp.float32),
      ))


_agg_bf_128 = _make_agg_bf(128)
_agg_bf_64 = _make_agg_bf(64)
_cnt_call = _make_cnt()


def _inv_deg(cnt_ref):
  cv = cnt_ref[0] + cnt_ref[1]
  return 1.0 / jnp.maximum(jnp.max(cv, axis=1, keepdims=True), 1.0)


def _dot(a, b):
  return jnp.dot(a, b, preferred_element_type=jnp.float32)


def _tc1_body(agg, cnt, xb, wl1, wr1, b1, wl2, wr2, b2, p2, r2):
  mean = (agg[0] + agg[1]) * _inv_deg(cnt)
  h = _dot(mean, wl1[...]) + _dot(xb[...], wr1[...]) + b1[...]
  h = jnp.maximum(h, 0.0)
  p2[...] = _dot(h, wl2[...])
  r2[...] = _dot(h, wr2[...]) + b2[...]


def _tc2_body(agg, cnt, r2, wl3, wr3, b3, p3, r3):
  h = jnp.maximum((agg[0] + agg[1]) * _inv_deg(cnt) + r2[...], 0.0)
  p3[...] = _dot(h, wl3[...])
  r3[...] = _dot(h, wr3[...]) + b3[...]


def _tc3_body(agg, cnt, r3, out, ls):
  o = (agg[0] + agg[1]) * _inv_deg(cnt) + r3[...]
  out[...] = o
  m = jnp.max(o, axis=1, keepdims=True)
  e = jnp.exp(o - m)
  ls[...] = o - m - jnp.log(jnp.sum(e, axis=1, keepdims=True))


def _row_spec(d):
  return pl.BlockSpec((ROWT, d), lambda i: (i, 0))


def _agg_spec(d):
  return pl.BlockSpec((NC, ROWT, d), lambda i: (0, i, 0))


def _full_spec(shape):
  return pl.BlockSpec(shape, lambda i: tuple(0 for _ in shape))


def _tc_call(body, in_specs, out_specs, out_shape):
  return pl.pallas_call(
      body, grid=(GRID,), in_specs=in_specs, out_specs=out_specs,
      out_shape=out_shape)


def kernel(x, edge_index, Wl1, Wr1, b1, Wl2, Wr2, b2, Wl3, Wr3, b3):
  src = edge_index[0].astype(jnp.int32)
  dst = edge_index[1].astype(jnp.int32)
  pad = EP - E
  src_p = jnp.concatenate([src, jnp.zeros((pad,), jnp.int32)])
  dst_p = jnp.concatenate([dst, jnp.full((pad,), DUMMY, jnp.int32)])
  src_p = src_p.reshape(TOTC, CHUNK)
  dst_p = dst_p.reshape(TOTC, CHUNK)

  z128 = jnp.zeros((NP, 128), jnp.float32)
  z64 = jnp.zeros((NP, 64), jnp.float32)
  zcnt = jnp.zeros((NP, CNTW), jnp.float32)
  ones_h = jnp.ones((CHUNK, CNTW), jnp.float32)
  b1r = b1.reshape(1, -1)
  b2r = b2.reshape(1, -1)
  b3r = b3.reshape(1, -1)

  cnt = _cnt_call(dst_p, zcnt, ones_h)
  agg1 = _agg_bf_128(_pack_bf16(x), src_p, dst_p, z128)

  p2, r2 = _tc_call(
      _tc1_body,
      [_agg_spec(128), _agg_spec(CNTW), _row_spec(128),
       _full_spec((128, 256)), _full_spec((128, 256)), _full_spec((1, 256)),
       _full_spec((256, 128)), _full_spec((256, 128)), _full_spec((1, 128))],
      [_row_spec(128), _row_spec(128)],
      [jax.ShapeDtypeStruct((N, 128), jnp.float32),
       jax.ShapeDtypeStruct((N, 128), jnp.float32)],
  )(agg1, cnt, x, Wl1, Wr1, b1r, Wl2, Wr2, b2r)

  agg2 = _agg_bf_128(_pack_bf16(p2), src_p, dst_p, z128)

  p3, r3 = _tc_call(
      _tc2_body,
      [_agg_spec(128), _agg_spec(CNTW), _row_spec(128),
       _full_spec((128, 64)), _full_spec((128, 64)), _full_spec((1, 64))],
      [_row_spec(64), _row_spec(64)],
      [jax.ShapeDtypeStruct((N, 64), jnp.float32),
       jax.ShapeDtypeStruct((N, 64), jnp.float32)],
  )(agg2, cnt, r2, Wl3, Wr3, b3r)

  agg3 = _agg_bf_64(_pack_bf16(p3), src_p, dst_p, z64)

  out, ls = _tc_call(
      _tc3_body,
      [_agg_spec(64), _agg_spec(CNTW), _row_spec(64)],
      [_row_spec(64), _row_spec(64)],
      [jax.ShapeDtypeStruct((N, 64), jnp.float32),
       jax.ShapeDtypeStruct((N, 64), jnp.float32)],
  )(agg3, cnt, r3)

  return (out, ls)
